# Initial kernel scaffold; baseline (speedup 1.0000x reference)
import jax
import jax.numpy as jnp
from jax import lax
from jax.experimental import pallas as pl
from jax.experimental.pallas import tpu as pltpu
from jax.experimental.pallas import tpu_sc as plsc

N_NODES = 50000
D = 128
FILTER = 10

NW = 32
GROUP = 32
IDX_PER_ROW = 12
B_PAD = 50176
ROWS_PER_W = B_PAD // NW
GROUPS_PER_W = ROWS_PER_W // GROUP
IDX_ROWS_PER_GROUP = GROUP * IDX_PER_ROW // 128

BM = 400


def _matmul_body(f_ref, w_ref, o_ref):
    o_ref[...] = lax.dot_general(
        f_ref[...], w_ref[0],
        dimension_numbers=(((1,), (0,)), ((), ())),
        preferred_element_type=jnp.float32,
    )


def _build_tables(features, w_stack):
    nblk = N_NODES // BM
    return pl.pallas_call(
        _matmul_body,
        grid=(nblk, 2),
        in_specs=[
            pl.BlockSpec((BM, D), lambda i, t: (i, 0)),
            pl.BlockSpec((1, D, D), lambda i, t: (t, 0, 0)),
        ],
        out_specs=pl.BlockSpec((BM, D), lambda i, t: (t * nblk + i, 0)),
        out_shape=jax.ShapeDtypeStruct((2 * N_NODES, D), jnp.float32),
    )(features, w_stack)


def _sc_body(tab_hbm, idx_hbm, out_hbm, idx_v, rows_v, out_v, sem):
    wid = lax.axis_index("s") * 2 + lax.axis_index("c")

    def group_body(g, carry):
        idx_base = wid * (GROUPS_PER_W * IDX_ROWS_PER_GROUP) + g * IDX_ROWS_PER_GROUP
        pltpu.sync_copy(idx_hbm.at[pl.ds(idx_base, IDX_ROWS_PER_GROUP)], idx_v)
        copies = [
            pltpu.async_copy(
                tab_hbm.at[idx_v.at[j]],
                rows_v.at[pl.ds(j * 128, 128)],
                sem,
            )
            for j in range(IDX_ROWS_PER_GROUP)
        ]
        for c in copies:
            c.wait()

        def row_body(r, carry2):
            base = r * IDX_PER_ROW
            for v in range(D // 16):
                sl = pl.ds(v * 16, 16)
                acc = rows_v[base, sl]
                for j in range(1, FILTER + 1):
                    acc = acc + rows_v[base + j, sl]
            out_v[r, sl] = jnp.maximum(acc, 0.0)
            return carry2

        lax.fori_loop(0, GROUP, row_body, 0)
        pltpu.sync_copy(
            out_v, out_hbm.at[pl.ds(wid * ROWS_PER_W + g * GROUP, GROUP)]
        )
        return carry

    lax.fori_loop(0, GROUPS_PER_W, group_body, 0)


def _sc_gather(tables, idx_hbm):
    mesh = plsc.VectorSubcoreMesh(core_axis_name="c", subcore_axis_name="s")
    return pl.kernel(
        _sc_body,
        out_type=jax.ShapeDtypeStruct((B_PAD, D), jnp.float32),
        mesh=mesh,
        scratch_types=[
            pltpu.VMEM((IDX_ROWS_PER_GROUP, 128), jnp.int32),
            pltpu.VMEM((GROUP * IDX_PER_ROW, D), jnp.float32),
            pltpu.VMEM((GROUP, D), jnp.float32),
            pltpu.SemaphoreType.DMA,
        ],
    )(tables, idx_hbm)


def kernel(nodes, neigh_idx, features, detaching_weight):
    w_top = detaching_weight[:D]
    w_bot = detaching_weight[D:] * (1.0 / FILTER)
    w_stack = jnp.stack([w_top, w_bot])
    tables = _build_tables(features, w_stack)

    nodes32 = nodes.astype(jnp.int32)
    idx12 = jnp.concatenate(
        [
            nodes32[:, None],
            neigh_idx.astype(jnp.int32) + N_NODES,
            jnp.zeros((N_NODES, 1), jnp.int32),
        ],
        axis=1,
    )
    idx_pad = jnp.pad(idx12, ((0, B_PAD - N_NODES), (0, 0)))
    idx_hbm = idx_pad.reshape(-1, 128)

    out_pad = _sc_gather(tables, idx_hbm)
    return out_pad[:N_NODES]


# speedup vs baseline: 2.9037x; 2.9037x over previous
import jax
import jax.numpy as jnp
from jax import lax
from jax.experimental import pallas as pl
from jax.experimental.pallas import tpu as pltpu
from jax.experimental.pallas import tpu_sc as plsc

N_NODES = 50000
D = 128
FILTER = 10

NW = 32
GROUP = 32
IDX_PER_ROW = 11
B_PAD = 50176
ROWS_PER_W = B_PAD // NW
GROUPS_PER_W = ROWS_PER_W // GROUP
IDX_PER_GROUP = GROUP * IDX_PER_ROW
GATHER_CHUNKS = [(0, 128), (128, 128), (256, 96)]

BM = 400


def _matmul_body(f_ref, w_ref, o_ref):
    o_ref[...] = lax.dot_general(
        f_ref[...], w_ref[0],
        dimension_numbers=(((1,), (0,)), ((), ())),
        preferred_element_type=jnp.float32,
    )


def _build_tables(features, w_stack):
    nblk = N_NODES // BM
    return pl.pallas_call(
        _matmul_body,
        grid=(nblk, 2),
        in_specs=[
            pl.BlockSpec((BM, D), lambda i, t: (i, 0)),
            pl.BlockSpec((1, D, D), lambda i, t: (t, 0, 0)),
        ],
        out_specs=pl.BlockSpec((BM, D), lambda i, t: (t * nblk + i, 0)),
        out_shape=jax.ShapeDtypeStruct((2 * N_NODES, D), jnp.float32),
    )(features, w_stack)


def _sc_body(tab_hbm, idx_hbm, out_hbm, idx_v, rows_v, out_v, sem):
    wid = lax.axis_index("s") * 2 + lax.axis_index("c")

    def group_body(g, carry):
        idx_base = (wid * ROWS_PER_W + g * GROUP) * IDX_PER_ROW
        pltpu.sync_copy(idx_hbm.at[pl.ds(idx_base, IDX_PER_GROUP)], idx_v)
        copies = [
            pltpu.async_copy(
                tab_hbm.at[idx_v.at[pl.ds(off, n)]],
                rows_v.at[pl.ds(off, n)],
                sem,
            )
            for off, n in GATHER_CHUNKS
        ]
        for c in copies:
            c.wait()

        def row_body(r, carry2):
            base = r * IDX_PER_ROW
            for v in range(D // 16):
                sl = pl.ds(v * 16, 16)
                acc = rows_v[base, sl]
                for j in range(1, FILTER + 1):
                    acc = acc + rows_v[base + j, sl]
                out_v[r, sl] = jnp.maximum(acc, 0.0)
            return carry2

        lax.fori_loop(0, GROUP, row_body, 0)
        pltpu.sync_copy(
            out_v, out_hbm.at[pl.ds(wid * ROWS_PER_W + g * GROUP, GROUP)]
        )
        return carry

    lax.fori_loop(0, GROUPS_PER_W, group_body, 0)


def _sc_gather(tables, idx_hbm):
    mesh = plsc.VectorSubcoreMesh(core_axis_name="c", subcore_axis_name="s")
    return pl.kernel(
        _sc_body,
        out_type=jax.ShapeDtypeStruct((B_PAD, D), jnp.float32),
        mesh=mesh,
        scratch_types=[
            pltpu.VMEM((IDX_PER_GROUP,), jnp.int32),
            pltpu.VMEM((IDX_PER_GROUP, D), jnp.float32),
            pltpu.VMEM((GROUP, D), jnp.float32),
            pltpu.SemaphoreType.DMA,
        ],
    )(tables, idx_hbm)


def kernel(nodes, neigh_idx, features, detaching_weight):
    w_top = detaching_weight[:D]
    w_bot = detaching_weight[D:] * (1.0 / FILTER)
    w_stack = jnp.stack([w_top, w_bot])
    tables = _build_tables(features, w_stack)

    nodes32 = nodes.astype(jnp.int32)
    idx11 = jnp.concatenate(
        [
            nodes32[:, None],
            neigh_idx.astype(jnp.int32) + N_NODES,
        ],
        axis=1,
    )
    idx_pad = jnp.pad(idx11, ((0, B_PAD - N_NODES), (0, 0)))
    idx_hbm = idx_pad.reshape(-1)

    out_pad = _sc_gather(tables, idx_hbm)
    return out_pad[:N_NODES]


# speedup vs baseline: 4.5567x; 1.5693x over previous
import jax
import jax.numpy as jnp
from jax import lax
from jax.experimental import pallas as pl
from jax.experimental.pallas import tpu as pltpu
from jax.experimental.pallas import tpu_sc as plsc

N_NODES = 50000
D = 128
FILTER = 10

NW = 32
GROUP = 112
IDX_PER_ROW = 11
B_PAD = 50176
ROWS_PER_W = B_PAD // NW
GROUPS_PER_W = ROWS_PER_W // GROUP
IDX_PER_GROUP = GROUP * IDX_PER_ROW

BM = 400


def _matmul_body(f_ref, w_ref, o_ref):
    o_ref[...] = lax.dot_general(
        f_ref[...], w_ref[0],
        dimension_numbers=(((1,), (0,)), ((), ())),
        preferred_element_type=jnp.float32,
    )


def _build_tables(features, w_stack):
    nblk = N_NODES // BM
    return pl.pallas_call(
        _matmul_body,
        grid=(nblk, 2),
        in_specs=[
            pl.BlockSpec((BM, D), lambda i, t: (i, 0)),
            pl.BlockSpec((1, D, D), lambda i, t: (t, 0, 0)),
        ],
        out_specs=pl.BlockSpec((BM, D), lambda i, t: (t * nblk + i, 0)),
        out_shape=jax.ShapeDtypeStruct((2 * N_NODES, D), jnp.float32),
    )(features, w_stack)


def _sc_body(tab_hbm, idx_hbm, out_hbm, idx_v, acc_v, sem):
    wid = lax.axis_index("s") * 2 + lax.axis_index("c")
    zeros16 = jnp.zeros((16,), jnp.float32)

    def group_body(g, carry):
        idx_base = (wid * GROUPS_PER_W + g) * IDX_PER_GROUP
        pltpu.sync_copy(idx_hbm.at[pl.ds(idx_base, IDX_PER_GROUP)], idx_v)

        def zero_body(r, c):
            for v in range(D // 16):
                acc_v[r, pl.ds(v * 16, 16)] = zeros16
            return c

        lax.fori_loop(0, GROUP, zero_body, 0)

        copies = [
            pltpu.async_copy(
                tab_hbm.at[idx_v.at[pl.ds(j * GROUP, GROUP)]],
                acc_v,
                sem,
                add=True,
            )
            for j in range(IDX_PER_ROW)
        ]
        for c in copies:
            c.wait()

        def relu_body(r, c):
            for v in range(D // 16):
                sl = pl.ds(v * 16, 16)
                acc_v[r, sl] = jnp.maximum(acc_v[r, sl], 0.0)
            return c

        lax.fori_loop(0, GROUP, relu_body, 0)
        pltpu.sync_copy(
            acc_v, out_hbm.at[pl.ds(wid * ROWS_PER_W + g * GROUP, GROUP)]
        )
        return carry

    lax.fori_loop(0, GROUPS_PER_W, group_body, 0)


def _sc_gather(tables, idx_hbm):
    mesh = plsc.VectorSubcoreMesh(core_axis_name="c", subcore_axis_name="s")
    return pl.kernel(
        _sc_body,
        out_type=jax.ShapeDtypeStruct((B_PAD, D), jnp.float32),
        mesh=mesh,
        scratch_types=[
            pltpu.VMEM((IDX_PER_GROUP,), jnp.int32),
            pltpu.VMEM((GROUP, D), jnp.float32),
            pltpu.SemaphoreType.DMA,
        ],
    )(tables, idx_hbm)


def kernel(nodes, neigh_idx, features, detaching_weight):
    w_top = detaching_weight[:D]
    w_bot = detaching_weight[D:] * (1.0 / FILTER)
    w_stack = jnp.stack([w_top, w_bot])
    tables = _build_tables(features, w_stack)

    nodes32 = nodes.astype(jnp.int32)
    idx11 = jnp.concatenate(
        [
            nodes32[:, None],
            neigh_idx.astype(jnp.int32) + N_NODES,
        ],
        axis=1,
    )
    idx_pad = jnp.pad(idx11, ((0, B_PAD - N_NODES), (0, 0)))
    idx_t = idx_pad.reshape(NW * GROUPS_PER_W, GROUP, IDX_PER_ROW)
    idx_t = jnp.swapaxes(idx_t, 1, 2)
    idx_hbm = idx_t.reshape(-1)

    out_pad = _sc_gather(tables, idx_hbm)
    return out_pad[:N_NODES]


# speedup vs baseline: 5.4520x; 1.1965x over previous
import jax
import jax.numpy as jnp
from jax import lax
from jax.experimental import pallas as pl
from jax.experimental.pallas import tpu as pltpu
from jax.experimental.pallas import tpu_sc as plsc

N_NODES = 50000
D = 128
FILTER = 10

NW = 32
GROUP = 112
IDX_PER_ROW = 11
B_PAD = 50176
ROWS_PER_W = B_PAD // NW
GROUPS_PER_W = ROWS_PER_W // GROUP
IDX_PER_GROUP = GROUP * IDX_PER_ROW

BM = 448


def _sc_body(feat_hbm, idx_hbm, out_hbm, idx_v, self_v, acc_v, sem):
    wid = lax.axis_index("s") * 2 + lax.axis_index("c")
    zeros16 = jnp.zeros((16,), jnp.float32)

    def group_body(g, carry):
        idx_base = (wid * GROUPS_PER_W + g) * IDX_PER_GROUP
        pltpu.sync_copy(idx_hbm.at[pl.ds(idx_base, IDX_PER_GROUP)], idx_v)

        def zero_body(r, c):
            for v in range(D // 16):
                acc_v[r, pl.ds(v * 16, 16)] = zeros16
            return c

        lax.fori_loop(0, GROUP, zero_body, 0)

        copies = [
            pltpu.async_copy(
                feat_hbm.at[idx_v.at[pl.ds(0, GROUP)]], self_v, sem
            )
        ] + [
            pltpu.async_copy(
                feat_hbm.at[idx_v.at[pl.ds(j * GROUP, GROUP)]],
                acc_v,
                sem,
                add=True,
            )
            for j in range(1, IDX_PER_ROW)
        ]
        for c in copies:
            c.wait()

        row0 = wid * ROWS_PER_W + g * GROUP
        pltpu.sync_copy(self_v, out_hbm.at[pl.ds(row0, GROUP)])
        pltpu.sync_copy(acc_v, out_hbm.at[pl.ds(B_PAD + row0, GROUP)])
        return carry

    lax.fori_loop(0, GROUPS_PER_W, group_body, 0)


def _sc_gather(features, idx_hbm):
    mesh = plsc.VectorSubcoreMesh(core_axis_name="c", subcore_axis_name="s")
    return pl.kernel(
        _sc_body,
        out_type=jax.ShapeDtypeStruct((2 * B_PAD, D), jnp.float32),
        mesh=mesh,
        scratch_types=[
            pltpu.VMEM((IDX_PER_GROUP,), jnp.int32),
            pltpu.VMEM((GROUP, D), jnp.float32),
            pltpu.VMEM((GROUP, D), jnp.float32),
            pltpu.SemaphoreType.DMA,
        ],
    )(features, idx_hbm)


def _matmul_body(s_ref, n_ref, w_ref, o_ref):
    ps = lax.dot_general(
        s_ref[...], w_ref[0],
        dimension_numbers=(((1,), (0,)), ((), ())),
        preferred_element_type=jnp.float32,
    )
    pn = lax.dot_general(
        n_ref[...], w_ref[1],
        dimension_numbers=(((1,), (0,)), ((), ())),
        preferred_element_type=jnp.float32,
    )
    o_ref[...] = jnp.maximum(ps + pn, 0.0)


def _fused_matmul(gathered, w_stack):
    nblk = B_PAD // BM
    return pl.pallas_call(
        _matmul_body,
        grid=(nblk,),
        in_specs=[
            pl.BlockSpec((BM, D), lambda i: (i, 0)),
            pl.BlockSpec((BM, D), lambda i: (nblk + i, 0)),
            pl.BlockSpec((2, D, D), lambda i: (0, 0, 0)),
        ],
        out_specs=pl.BlockSpec((BM, D), lambda i: (i, 0)),
        out_shape=jax.ShapeDtypeStruct((B_PAD, D), jnp.float32),
    )(gathered, gathered, w_stack)


def kernel(nodes, neigh_idx, features, detaching_weight):
    w_top = detaching_weight[:D]
    w_bot = detaching_weight[D:] * (1.0 / FILTER)
    w_stack = jnp.stack([w_top, w_bot])

    nodes32 = nodes.astype(jnp.int32)
    idx11 = jnp.concatenate(
        [nodes32[:, None], neigh_idx.astype(jnp.int32)], axis=1
    )
    idx_pad = jnp.pad(idx11, ((0, B_PAD - N_NODES), (0, 0)))
    idx_t = jnp.swapaxes(
        idx_pad.reshape(NW * GROUPS_PER_W, GROUP, IDX_PER_ROW), 1, 2
    )
    idx_hbm = idx_t.reshape(-1)

    gathered = _sc_gather(features, idx_hbm)
    out_pad = _fused_matmul(gathered, w_stack)
    return out_pad[:N_NODES]
